# baseline (speedup 1.0000x reference)
import functools

import jax
import jax.numpy as jnp
from jax import lax
from jax.experimental import pallas as pl
from jax.experimental.pallas import tpu as pltpu
from jax.experimental.pallas import tpu_sc as plsc

f32 = jnp.float32
i32 = jnp.int32

_NU = 25000
_NN = 50000
_D = 64
_E = 800000
_NC = 2
_NS = 16
_L = 16
_NQ = 4
_QSZ = _NN // _NQ
_TRASH = _QSZ
_ACC_ROWS = _QSZ + 8
_EPP = _E // (_NC * _NS)
_CAP = 26624
_PC = 256
_NPC = _EPP // _PC
_PTAIL = _EPP - _NPC * _PC
_MC = 1024
_NSUB = _MC // 128
_TQ = 832
_RQ = 32

_mesh = plsc.VectorSubcoreMesh(core_axis_name="c", subcore_axis_name="s")
_cparams = pltpu.CompilerParams(needs_layout_passes=False,
                                use_tc_tiling_on_sc=False)
_iota16 = lambda: lax.broadcasted_iota(i32, (_L,), 0)


def _rsqrt16(dv):
  ii = plsc.bitcast(dv, i32)
  ii = jnp.full((_L,), 0x5F3759DF, i32) - lax.shift_right_arithmetic(ii, 1)
  y = plsc.bitcast(ii, f32)
  for _ in range(3):
    y = y * (1.5 - 0.5 * dv * y * y)
  return jnp.where(dv > 0.0, y, 0.0)


def _zero_rows(buf, n):
  w = buf.shape[1]

  def body(i, _):
    for q in range(w // _L):
      buf[i, pl.ds(q * _L, _L)] = jnp.zeros((_L,), f32)
    return 0

  lax.fori_loop(0, n, body, 0)


def _count_for(cntbuf, q):
  v = cntbuf[pl.ds(0, _L)]
  return jnp.sum(jnp.where(_iota16() == q, v, 0))


def _part_body(rows_h, cols_h, colp_h, dstp_h, cnt_h,
               rbuf, cbuf, pend_c, pend_d, cntbuf):
  c = lax.axis_index("c")
  s = lax.axis_index("s")
  e0 = (s * _NC + c) * _EPP

  def do_group(offs, g, valid_mask):
    rv = rbuf[pl.ds(g * _L, _L)]
    cv = cbuf[pl.ds(g * _L, _L)]
    offs = list(offs)
    for q in range(_NQ):
      m = (rv >= q * _QSZ) & (rv < (q + 1) * _QSZ)
      if valid_mask is not None:
        m = m & valid_mask
      loc = rv - q * _QSZ
      plsc.store_compressed(pend_d.at[q, pl.ds(offs[q], _L)], loc, mask=m)
      plsc.store_compressed(pend_c.at[q, pl.ds(offs[q], _L)], cv, mask=m)
      offs[q] = offs[q] + plsc.all_reduce_population_count(m)[0]
    return tuple(offs)

  def flush_q(q, off, cnt):
    def yes(o, n):
      na = pl.multiple_of(n, 1024)
      pltpu.sync_copy(pend_c.at[q, pl.ds(0, _MC)],
                      colp_h.at[c, s, q, pl.ds(na, _MC)])
      pltpu.sync_copy(pend_d.at[q, pl.ds(0, _MC)],
                      dstp_h.at[c, s, q, pl.ds(na, _MC)])
      for g in range(_PC // _L):
        pend_c[q, pl.ds(g * _L, _L)] = pend_c[q, pl.ds(_MC + g * _L, _L)]
        pend_d[q, pl.ds(g * _L, _L)] = pend_d[q, pl.ds(_MC + g * _L, _L)]
      return o - _MC, n + _MC

    return lax.cond(off >= _MC, yes, lambda o, n: (o, n), off, cnt)

  def chunk(j, carry):
    pltpu.sync_copy(rows_h.at[pl.ds(e0 + j * _PC, _PC)], rbuf)
    pltpu.sync_copy(cols_h.at[pl.ds(e0 + j * _PC, _PC)], cbuf)
    offs = carry[:_NQ]
    cnts = carry[_NQ:]
    for g in range(_PC // _L):
      offs = do_group(offs, g, None)
    offs = list(offs)
    cnts = list(cnts)
    for q in range(_NQ):
      offs[q], cnts[q] = flush_q(q, offs[q], cnts[q])
    return tuple(offs) + tuple(cnts)

  zero = jnp.asarray(0, i32)
  carry = lax.fori_loop(0, _NPC, chunk, (zero,) * (2 * _NQ))
  pltpu.sync_copy(rows_h.at[pl.ds(e0 + _NPC * _PC, 176)],
                  rbuf.at[pl.ds(0, 176)])
  pltpu.sync_copy(cols_h.at[pl.ds(e0 + _NPC * _PC, 176)],
                  cbuf.at[pl.ds(0, 176)])
  offs = carry[:_NQ]
  cnts = list(carry[_NQ:])
  for g in range(_PTAIL // _L):
    offs = do_group(offs, g, None)
  tail_mask = _iota16() < (_PTAIL - (_PTAIL // _L) * _L)
  offs = do_group(offs, _PTAIL // _L, tail_mask)
  offs = list(offs)
  for q in range(_NQ):
    offs[q], cnts[q] = flush_q(q, offs[q], cnts[q])
  nv = jnp.zeros((_L,), i32)
  for q in range(_NQ):
    na = pl.multiple_of(cnts[q], 1024)
    pltpu.sync_copy(pend_c.at[q, pl.ds(0, _MC)],
                    colp_h.at[c, s, q, pl.ds(na, _MC)])
    pltpu.sync_copy(pend_d.at[q, pl.ds(0, _MC)],
                    dstp_h.at[c, s, q, pl.ds(na, _MC)])
    nv = jnp.where(_iota16() == q, jnp.full((_L,), cnts[q] + offs[q], i32), nv)
  cntbuf[pl.ds(0, _L)] = nv
  pltpu.sync_copy(cntbuf, cnt_h.at[c, s])


def _make_part():
  return pl.kernel(
      _part_body,
      out_type=(jax.ShapeDtypeStruct((_NC, _NS, _NQ, _CAP), i32),
                jax.ShapeDtypeStruct((_NC, _NS, _NQ, _CAP), i32),
                jax.ShapeDtypeStruct((_NC, _NS, _L), i32)),
      mesh=_mesh,
      compiler_params=_cparams,
      scratch_types=[
          pltpu.VMEM((_PC,), i32),
          pltpu.VMEM((_PC,), i32),
          pltpu.VMEM((_NQ, _MC + 2 * _PC), i32),
          pltpu.VMEM((_NQ, _MC + 2 * _PC), i32),
          pltpu.VMEM((_L,), i32),
      ],
  )


def _load_ibuf(dstp_h, c2, s2, q, off, dbuf, ibuf, sanitize=None):
  pltpu.sync_copy(dstp_h.at[c2, s2, q, pl.ds(off, _MC)], dbuf)
  if sanitize is None:
    for g in range(_MC // _L):
      ibuf[g // 8, pl.ds((g % 8) * _L, _L)] = dbuf[pl.ds(g * _L, _L)]
  else:
    n, = sanitize
    for g in range(_MC // _L):
      eid = off + g * _L + _iota16()
      iv = dbuf[pl.ds(g * _L, _L)]
      ibuf[g // 8, pl.ds((g % 8) * _L, _L)] = jnp.where(eid < n, iv, _TRASH)


def _drain_cover(s):
  return jnp.minimum(s * _TQ, _QSZ - _TQ)


def _deg_body(dstp_h, cnt_h, x0_h, dinv_h, g0_h,
              ones_v, dibuf, ibuf, cntbuf, dbuf, dvbuf, xbuf, zb16, deg_sp,
              sems):
  c = lax.axis_index("c")
  s = lax.axis_index("s")

  def ones_fill(i, _):
    ones_v[i, :] = jnp.full((_L,), 1.0, f32)
    return 0

  lax.fori_loop(0, 128, ones_fill, 0)
  _zero_rows(zb16, _RQ)
  z0 = _drain_cover(s)

  def zloop(k, _):
    pltpu.sync_copy(zb16, deg_sp.at[pl.ds(z0 + k * _RQ, _RQ), :])
    return 0

  lax.fori_loop(0, _TQ // _RQ, zloop, 0)

  @pl.when(s == 0)
  def _():
    pltpu.sync_copy(zb16.at[pl.ds(0, 8), :], deg_sp.at[pl.ds(_QSZ, 8), :])

  plsc.subcore_barrier()

  def scatter8():
    descs = [
        pltpu.async_copy(ones_v, deg_sp.at[ibuf.at[k]], sems, add=True)
        for k in range(_NSUB)
    ]
    for d in descs:
      d.wait()

  for sweep in range(2):
    q = 2 * sweep + c
    for wpc in range(_NC):
      pltpu.sync_copy(cnt_h.at[wpc, s], cntbuf)
      n = _count_for(cntbuf, q)
      nmac = n // _MC
      t = n - nmac * _MC

      def mac(j, _):
        _load_ibuf(dstp_h, wpc, s, q, pl.multiple_of(j * _MC, 1024),
                   dibuf, ibuf)
        scatter8()
        return 0

      lax.fori_loop(0, nmac, mac, 0)

      @pl.when(t > 0)
      def _():
        _load_ibuf(dstp_h, wpc, s, q, pl.multiple_of(nmac * _MC, 1024),
                   dibuf, ibuf, sanitize=(n,))
        scatter8()

    plsc.subcore_barrier()
    r0 = _drain_cover(s)

    def dchunk(k, _):
      row0 = r0 + k * _RQ
      nbase = q * _QSZ + row0
      pltpu.sync_copy(deg_sp.at[pl.ds(row0, _RQ), :], dbuf)
      pltpu.sync_copy(x0_h.at[pl.ds(nbase, _RQ), :], xbuf)
      for i in range(_RQ):
        dvv = _rsqrt16(dbuf[i, :])
        dvbuf[i, :] = dvv
        for w in range(_D // _L):
          xbuf[i, pl.ds(w * _L, _L)] = xbuf[i, pl.ds(w * _L, _L)] * dvv
      pltpu.sync_copy(dvbuf, dinv_h.at[pl.ds(nbase, _RQ), :])
      pltpu.sync_copy(xbuf, g0_h.at[pl.ds(nbase, _RQ), :])
      return 0

    lax.fori_loop(0, _TQ // _RQ, dchunk, 0)
    if sweep == 0:
      plsc.subcore_barrier()
      lax.fori_loop(0, _TQ // _RQ, zloop, 0)
    plsc.subcore_barrier()


def _make_deg():
  return pl.kernel(
      _deg_body,
      out_type=(jax.ShapeDtypeStruct((_NN, _L), f32),
                jax.ShapeDtypeStruct((_NN, _D), f32)),
      mesh=_mesh,
      compiler_params=_cparams,
      scratch_types=[
          pltpu.VMEM((128, _L), f32),
          pltpu.VMEM((_MC,), i32),
          pltpu.VMEM((_NSUB, 128), i32),
          pltpu.VMEM((_L,), i32),
          pltpu.VMEM((_RQ, _L), f32),
          pltpu.VMEM((_RQ, _L), f32),
          pltpu.VMEM((_RQ, _D), f32),
          pltpu.VMEM((_RQ, _L), f32),
          pltpu.VMEM_SHARED((_ACC_ROWS, _L), f32),
          pltpu.SemaphoreType.DMA,
      ],
  )


def _layer_body(last, colp_h, dstp_h, cnt_h, dinv_h, g_h, sum_h, *refs):
  if last:
    (out_h, cbuf, dibuf, ibuf, cntbuf, gbuf, abuf, dvbuf, sbuf, zbuf, acc_sp,
     semg, sems) = refs
    go_h = None
  else:
    (out_h, go_h, cbuf, dibuf, ibuf, cntbuf, gbuf, abuf, dvbuf, sbuf, zbuf,
     acc_sp, semg, sems) = refs

  c = lax.axis_index("c")
  s = lax.axis_index("s")

  _zero_rows(zbuf, _RQ)
  z0 = _drain_cover(s)

  def zloop(k, _):
    pltpu.sync_copy(zbuf, acc_sp.at[pl.ds(z0 + k * _RQ, _RQ), :])
    return 0

  lax.fori_loop(0, _TQ // _RQ, zloop, 0)

  @pl.when(s == 0)
  def _():
    pltpu.sync_copy(zbuf.at[pl.ds(0, 8), :], acc_sp.at[pl.ds(_QSZ, 8), :])

  plsc.subcore_barrier()

  def run_macro():
    sd = []
    for p in range(_NSUB // 2):
      gd = [
          pltpu.async_copy(g_h.at[cbuf.at[pl.ds(k * 128, 128)]],
                           gbuf.at[pl.ds(k * 128, 128), :], semg)
          for k in (2 * p, 2 * p + 1)
      ]
      for d in gd:
        d.wait()
      for k in (2 * p, 2 * p + 1):
        sd.append(pltpu.async_copy(gbuf.at[pl.ds(k * 128, 128), :],
                                   acc_sp.at[ibuf.at[k]], sems, add=True))
    for d in sd:
      d.wait()

  for sweep in range(2):
    q = 2 * sweep + c
    for wpc in range(_NC):
      pltpu.sync_copy(cnt_h.at[wpc, s], cntbuf)
      n = _count_for(cntbuf, q)
      nmac = n // _MC
      t = n - nmac * _MC

      def mac(j, _):
        ja = pl.multiple_of(j * _MC, 1024)
        pltpu.sync_copy(colp_h.at[wpc, s, q, pl.ds(ja, _MC)], cbuf)
        _load_ibuf(dstp_h, wpc, s, q, ja, dibuf, ibuf)
        run_macro()
        return 0

      lax.fori_loop(0, nmac, mac, 0)

      @pl.when(t > 0)
      def _():
        off = pl.multiple_of(nmac * _MC, 1024)
        pltpu.sync_copy(colp_h.at[wpc, s, q, pl.ds(off, _MC)], cbuf)
        _load_ibuf(dstp_h, wpc, s, q, off, dibuf, ibuf, sanitize=(n,))
        for g in range(_MC // _L):
          eid = off + g * _L + _iota16()
          cvv = cbuf[pl.ds(g * _L, _L)]
          cbuf[pl.ds(g * _L, _L)] = jnp.where(eid < n, cvv, 0)
        run_macro()

    plsc.subcore_barrier()
    r0 = _drain_cover(s)

    def dchunk(k, _):
      row0 = r0 + k * _RQ
      nbase = q * _QSZ + row0
      pltpu.sync_copy(acc_sp.at[pl.ds(row0, _RQ), :], abuf)
      pltpu.sync_copy(sum_h.at[pl.ds(nbase, _RQ), :], sbuf)
      pltpu.sync_copy(dinv_h.at[pl.ds(nbase, _RQ), :], dvbuf)

      def row(i, _):
        dvv = dvbuf[i, :]
        for w in range(_D // _L):
          xv = abuf[i, pl.ds(w * _L, _L)] * dvv
          sv = sbuf[i, pl.ds(w * _L, _L)] + xv
          if last:
            sbuf[i, pl.ds(w * _L, _L)] = sv * 0.25
          else:
            sbuf[i, pl.ds(w * _L, _L)] = sv
            abuf[i, pl.ds(w * _L, _L)] = xv * dvv
        return 0

      lax.fori_loop(0, _RQ, row, 0)
      pltpu.sync_copy(sbuf, out_h.at[pl.ds(nbase, _RQ), :])
      if not last:
        pltpu.sync_copy(abuf, go_h.at[pl.ds(nbase, _RQ), :])
      return 0

    lax.fori_loop(0, _TQ // _RQ, dchunk, 0)
    if sweep == 0:
      plsc.subcore_barrier()
      lax.fori_loop(0, _TQ // _RQ, zloop, 0)
      plsc.subcore_barrier()


def _make_layer(last):
  if last:
    outs = jax.ShapeDtypeStruct((_NN, _D), f32)
  else:
    outs = (jax.ShapeDtypeStruct((_NN, _D), f32),
            jax.ShapeDtypeStruct((_NN, _D), f32))
  return pl.kernel(
      functools.partial(_layer_body, last),
      out_type=outs,
      mesh=_mesh,
      compiler_params=_cparams,
      scratch_types=[
          pltpu.VMEM((_MC,), i32),
          pltpu.VMEM((_MC,), i32),
          pltpu.VMEM((_NSUB, 128), i32),
          pltpu.VMEM((_L,), i32),
          pltpu.VMEM((_MC, _D), f32),
          pltpu.VMEM((_RQ, _D), f32),
          pltpu.VMEM((_RQ, _L), f32),
          pltpu.VMEM((_RQ, _D), f32),
          pltpu.VMEM((_RQ, _D), f32),
          pltpu.VMEM_SHARED((_ACC_ROWS, _D), f32),
          pltpu.SemaphoreType.DMA,
          pltpu.SemaphoreType.DMA,
      ],
  )


_part_kernel = _make_part()
_deg_kernel = _make_deg()
_layer_kernel = _make_layer(False)
_layer_kernel_last = _make_layer(True)


def kernel(user_emb, item_emb, edge_index):
  x0 = jnp.concatenate([user_emb, item_emb], axis=0)
  rows = edge_index[0]
  cols = edge_index[1]
  colp, dstp, cnt = _part_kernel(rows, cols)
  dinv, g0 = _deg_kernel(dstp, cnt, x0)
  s1, g1 = _layer_kernel(colp, dstp, cnt, dinv, g0, x0)
  s2, g2 = _layer_kernel(colp, dstp, cnt, dinv, g1, s1)
  out = _layer_kernel_last(colp, dstp, cnt, dinv, g2, s2)
  return (out[:_NU], out[_NU:])


# speedup vs baseline: 1.0082x; 1.0082x over previous
import functools

import jax
import jax.numpy as jnp
from jax import lax
from jax.experimental import pallas as pl
from jax.experimental.pallas import tpu as pltpu
from jax.experimental.pallas import tpu_sc as plsc

f32 = jnp.float32
i32 = jnp.int32

_NU = 25000
_NN = 50000
_D = 64
_E = 800000
_NC = 2
_NS = 16
_L = 16
_NQ = 4
_QSZ = _NN // _NQ
_TRASH = _QSZ
_ACC_ROWS = _QSZ + 8
_EPP = _E // (_NC * _NS)
_CAP = 26624
_PC = 256
_NPC = _EPP // _PC
_PTAIL = _EPP - _NPC * _PC
_MC = 1024
_NSUB = _MC // 128
_TQ = 832
_RQ = 32

_mesh = plsc.VectorSubcoreMesh(core_axis_name="c", subcore_axis_name="s")
_cparams = pltpu.CompilerParams(needs_layout_passes=False,
                                use_tc_tiling_on_sc=False)
_iota16 = lambda: lax.broadcasted_iota(i32, (_L,), 0)


def _rsqrt16(dv):
  ii = plsc.bitcast(dv, i32)
  ii = jnp.full((_L,), 0x5F3759DF, i32) - lax.shift_right_arithmetic(ii, 1)
  y = plsc.bitcast(ii, f32)
  for _ in range(3):
    y = y * (1.5 - 0.5 * dv * y * y)
  return jnp.where(dv > 0.0, y, 0.0)


def _zero_rows(buf, n):
  w = buf.shape[1]

  def body(i, _):
    for q in range(w // _L):
      buf[i, pl.ds(q * _L, _L)] = jnp.zeros((_L,), f32)
    return 0

  lax.fori_loop(0, n, body, 0)


def _count_for(cntbuf, q):
  v = cntbuf[pl.ds(0, _L)]
  return jnp.sum(jnp.where(_iota16() == q, v, 0))


def _part_body(rows_h, cols_h, colp_h, dstp_h, cnt_h,
               rbuf, cbuf, pend_c, pend_d, cntbuf):
  c = lax.axis_index("c")
  s = lax.axis_index("s")
  e0 = (s * _NC + c) * _EPP

  def do_group(offs, g, valid_mask):
    rv = rbuf[pl.ds(g * _L, _L)]
    cv = cbuf[pl.ds(g * _L, _L)]
    offs = list(offs)
    for q in range(_NQ):
      m = (rv >= q * _QSZ) & (rv < (q + 1) * _QSZ)
      if valid_mask is not None:
        m = m & valid_mask
      loc = rv - q * _QSZ
      plsc.store_compressed(pend_d.at[q, pl.ds(offs[q], _L)], loc, mask=m)
      plsc.store_compressed(pend_c.at[q, pl.ds(offs[q], _L)], cv, mask=m)
      offs[q] = offs[q] + plsc.all_reduce_population_count(m)[0]
    return tuple(offs)

  def flush_q(q, off, cnt):
    def yes(o, n):
      na = pl.multiple_of(n, 1024)
      pltpu.sync_copy(pend_c.at[q, pl.ds(0, _MC)],
                      colp_h.at[c, s, q, pl.ds(na, _MC)])
      pltpu.sync_copy(pend_d.at[q, pl.ds(0, _MC)],
                      dstp_h.at[c, s, q, pl.ds(na, _MC)])
      for g in range(_PC // _L):
        pend_c[q, pl.ds(g * _L, _L)] = pend_c[q, pl.ds(_MC + g * _L, _L)]
        pend_d[q, pl.ds(g * _L, _L)] = pend_d[q, pl.ds(_MC + g * _L, _L)]
      return o - _MC, n + _MC

    return lax.cond(off >= _MC, yes, lambda o, n: (o, n), off, cnt)

  def chunk(j, carry):
    pltpu.sync_copy(rows_h.at[pl.ds(e0 + j * _PC, _PC)], rbuf)
    pltpu.sync_copy(cols_h.at[pl.ds(e0 + j * _PC, _PC)], cbuf)
    offs = carry[:_NQ]
    cnts = carry[_NQ:]
    for g in range(_PC // _L):
      offs = do_group(offs, g, None)
    offs = list(offs)
    cnts = list(cnts)
    for q in range(_NQ):
      offs[q], cnts[q] = flush_q(q, offs[q], cnts[q])
    return tuple(offs) + tuple(cnts)

  zero = jnp.asarray(0, i32)
  carry = lax.fori_loop(0, _NPC, chunk, (zero,) * (2 * _NQ))
  pltpu.sync_copy(rows_h.at[pl.ds(e0 + _NPC * _PC, 176)],
                  rbuf.at[pl.ds(0, 176)])
  pltpu.sync_copy(cols_h.at[pl.ds(e0 + _NPC * _PC, 176)],
                  cbuf.at[pl.ds(0, 176)])
  offs = carry[:_NQ]
  cnts = list(carry[_NQ:])
  for g in range(_PTAIL // _L):
    offs = do_group(offs, g, None)
  tail_mask = _iota16() < (_PTAIL - (_PTAIL // _L) * _L)
  offs = do_group(offs, _PTAIL // _L, tail_mask)
  offs = list(offs)
  for q in range(_NQ):
    offs[q], cnts[q] = flush_q(q, offs[q], cnts[q])
  nv = jnp.zeros((_L,), i32)
  for q in range(_NQ):
    na = pl.multiple_of(cnts[q], 1024)
    pltpu.sync_copy(pend_c.at[q, pl.ds(0, _MC)],
                    colp_h.at[c, s, q, pl.ds(na, _MC)])
    pltpu.sync_copy(pend_d.at[q, pl.ds(0, _MC)],
                    dstp_h.at[c, s, q, pl.ds(na, _MC)])
    nv = jnp.where(_iota16() == q, jnp.full((_L,), cnts[q] + offs[q], i32), nv)
  cntbuf[pl.ds(0, _L)] = nv
  pltpu.sync_copy(cntbuf, cnt_h.at[c, s])


def _make_part():
  return pl.kernel(
      _part_body,
      out_type=(jax.ShapeDtypeStruct((_NC, _NS, _NQ, _CAP), i32),
                jax.ShapeDtypeStruct((_NC, _NS, _NQ, _CAP), i32),
                jax.ShapeDtypeStruct((_NC, _NS, _L), i32)),
      mesh=_mesh,
      compiler_params=_cparams,
      scratch_types=[
          pltpu.VMEM((_PC,), i32),
          pltpu.VMEM((_PC,), i32),
          pltpu.VMEM((_NQ, _MC + 2 * _PC), i32),
          pltpu.VMEM((_NQ, _MC + 2 * _PC), i32),
          pltpu.VMEM((_L,), i32),
      ],
  )


def _load_ibuf(dstp_h, c2, s2, q, off, dbuf, ibuf, sanitize=None, pad=_TRASH):
  pltpu.sync_copy(dstp_h.at[c2, s2, q, pl.ds(off, _MC)], dbuf)
  if sanitize is None:
    for g in range(_MC // _L):
      ibuf[g // 8, pl.ds((g % 8) * _L, _L)] = dbuf[pl.ds(g * _L, _L)]
  else:
    n, = sanitize
    for g in range(_MC // _L):
      eid = off + g * _L + _iota16()
      iv = dbuf[pl.ds(g * _L, _L)]
      ibuf[g // 8, pl.ds((g % 8) * _L, _L)] = jnp.where(eid < n, iv, pad)


def _drain_cover(s):
  return jnp.minimum(s * _TQ, _QSZ - _TQ)


def _deg_body(dstp_h, cnt_h, x0_h, dinv_h, g0_h,
              ones_v, dibuf, ibuf, cntbuf, dbuf, dvbuf, xbuf, zb16, deg_sp,
              sems):
  c = lax.axis_index("c")
  s = lax.axis_index("s")

  def ones_fill(i, _):
    ones_v[i, :] = jnp.full((_L,), 1.0, f32)
    return 0

  lax.fori_loop(0, 128, ones_fill, 0)
  _zero_rows(zb16, _RQ)
  z0 = _drain_cover(s)

  def zloop(k, _):
    pltpu.sync_copy(zb16, deg_sp.at[pl.ds(z0 + k * _RQ, _RQ), :])
    return 0

  lax.fori_loop(0, _TQ // _RQ, zloop, 0)

  @pl.when(s == 0)
  def _():
    pltpu.sync_copy(zb16.at[pl.ds(0, 8), :], deg_sp.at[pl.ds(_QSZ, 8), :])

  plsc.subcore_barrier()

  def scatter8():
    descs = [
        pltpu.async_copy(ones_v, deg_sp.at[ibuf.at[k]], sems, add=True)
        for k in range(_NSUB)
    ]
    for d in descs:
      d.wait()

  for sweep in range(2):
    q = 2 * sweep + c
    for wpc in range(_NC):
      pltpu.sync_copy(cnt_h.at[wpc, s], cntbuf)
      n = _count_for(cntbuf, q)
      nmac = n // _MC
      t = n - nmac * _MC

      def mac(j, _):
        _load_ibuf(dstp_h, wpc, s, q, pl.multiple_of(j * _MC, 1024),
                   dibuf, ibuf)
        scatter8()
        return 0

      lax.fori_loop(0, nmac, mac, 0)

      @pl.when(t > 0)
      def _():
        _load_ibuf(dstp_h, wpc, s, q, pl.multiple_of(nmac * _MC, 1024),
                   dibuf, ibuf, sanitize=(n,))
        scatter8()

    plsc.subcore_barrier()
    r0 = _drain_cover(s)

    def dchunk(k, _):
      row0 = r0 + k * _RQ
      nbase = q * _QSZ + row0
      pltpu.sync_copy(deg_sp.at[pl.ds(row0, _RQ), :], dbuf)
      pltpu.sync_copy(x0_h.at[pl.ds(nbase, _RQ), :], xbuf)
      for i in range(_RQ):
        dvv = _rsqrt16(dbuf[i, :])
        dvbuf[i, :] = dvv
        for w in range(_D // _L):
          xbuf[i, pl.ds(w * _L, _L)] = xbuf[i, pl.ds(w * _L, _L)] * dvv
      pltpu.sync_copy(dvbuf, dinv_h.at[pl.ds(nbase, _RQ), :])
      pltpu.sync_copy(xbuf, g0_h.at[pl.ds(nbase, _RQ), :])
      return 0

    lax.fori_loop(0, _TQ // _RQ, dchunk, 0)
    if sweep == 0:
      plsc.subcore_barrier()
      lax.fori_loop(0, _TQ // _RQ, zloop, 0)
    plsc.subcore_barrier()


def _make_deg():
  return pl.kernel(
      _deg_body,
      out_type=(jax.ShapeDtypeStruct((_NN, _L), f32),
                jax.ShapeDtypeStruct((_NN, _D), f32)),
      mesh=_mesh,
      compiler_params=_cparams,
      scratch_types=[
          pltpu.VMEM((128, _L), f32),
          pltpu.VMEM((_MC,), i32),
          pltpu.VMEM((_NSUB, 128), i32),
          pltpu.VMEM((_L,), i32),
          pltpu.VMEM((_RQ, _L), f32),
          pltpu.VMEM((_RQ, _L), f32),
          pltpu.VMEM((_RQ, _D), f32),
          pltpu.VMEM((_RQ, _L), f32),
          pltpu.VMEM_SHARED((_ACC_ROWS, _L), f32),
          pltpu.SemaphoreType.DMA,
      ],
  )


def _layer_body(last, colp_h, dstp_h, cnt_h, dinv_h, g_h, sum_h, *refs):
  if last:
    (out_h, cibuf, dibuf, ibuf, cntbuf, gbuf, abuf, dvbuf, sbuf, zbuf, acc_sp,
     semg, sems) = refs
    go_h = None
  else:
    (out_h, go_h, cibuf, dibuf, ibuf, cntbuf, gbuf, abuf, dvbuf, sbuf, zbuf,
     acc_sp, semg, sems) = refs

  c = lax.axis_index("c")
  s = lax.axis_index("s")

  _zero_rows(zbuf, _RQ)
  z0 = _drain_cover(s)

  def zloop(k, _):
    pltpu.sync_copy(zbuf, acc_sp.at[pl.ds(z0 + k * _RQ, _RQ), :])
    return 0

  lax.fori_loop(0, _TQ // _RQ, zloop, 0)

  @pl.when(s == 0)
  def _():
    pltpu.sync_copy(zbuf.at[pl.ds(0, 8), :], acc_sp.at[pl.ds(_QSZ, 8), :])

  plsc.subcore_barrier()

  def run_macro():
    gd = [
        pltpu.async_copy(g_h.at[cibuf.at[k]],
                         gbuf.at[pl.ds(k * 128, 128), :], semg)
        for k in range(_NSUB)
    ]
    sd = []
    for k in range(_NSUB):
      gd[k].wait()
      sd.append(pltpu.async_copy(gbuf.at[pl.ds(k * 128, 128), :],
                                 acc_sp.at[ibuf.at[k]], sems, add=True))
    for d in sd:
      d.wait()

  for sweep in range(2):
    q = 2 * sweep + c
    for wpc in range(_NC):
      pltpu.sync_copy(cnt_h.at[wpc, s], cntbuf)
      n = _count_for(cntbuf, q)
      nmac = n // _MC
      t = n - nmac * _MC

      def mac(j, _):
        ja = pl.multiple_of(j * _MC, 1024)
        _load_ibuf(colp_h, wpc, s, q, ja, dibuf, cibuf)
        _load_ibuf(dstp_h, wpc, s, q, ja, dibuf, ibuf)
        run_macro()
        return 0

      lax.fori_loop(0, nmac, mac, 0)

      @pl.when(t > 0)
      def _():
        off = pl.multiple_of(nmac * _MC, 1024)
        _load_ibuf(colp_h, wpc, s, q, off, dibuf, cibuf, sanitize=(n,), pad=0)
        _load_ibuf(dstp_h, wpc, s, q, off, dibuf, ibuf, sanitize=(n,))
        run_macro()

    plsc.subcore_barrier()
    r0 = _drain_cover(s)

    def dchunk(k, _):
      row0 = r0 + k * _RQ
      nbase = q * _QSZ + row0
      pltpu.sync_copy(acc_sp.at[pl.ds(row0, _RQ), :], abuf)
      pltpu.sync_copy(sum_h.at[pl.ds(nbase, _RQ), :], sbuf)
      pltpu.sync_copy(dinv_h.at[pl.ds(nbase, _RQ), :], dvbuf)

      def row(i, _):
        dvv = dvbuf[i, :]
        for w in range(_D // _L):
          xv = abuf[i, pl.ds(w * _L, _L)] * dvv
          sv = sbuf[i, pl.ds(w * _L, _L)] + xv
          if last:
            sbuf[i, pl.ds(w * _L, _L)] = sv * 0.25
          else:
            sbuf[i, pl.ds(w * _L, _L)] = sv
            abuf[i, pl.ds(w * _L, _L)] = xv * dvv
        return 0

      lax.fori_loop(0, _RQ, row, 0)
      pltpu.sync_copy(sbuf, out_h.at[pl.ds(nbase, _RQ), :])
      if not last:
        pltpu.sync_copy(abuf, go_h.at[pl.ds(nbase, _RQ), :])
      return 0

    lax.fori_loop(0, _TQ // _RQ, dchunk, 0)
    if sweep == 0:
      plsc.subcore_barrier()
      lax.fori_loop(0, _TQ // _RQ, zloop, 0)
      plsc.subcore_barrier()


def _make_layer(last):
  if last:
    outs = jax.ShapeDtypeStruct((_NN, _D), f32)
  else:
    outs = (jax.ShapeDtypeStruct((_NN, _D), f32),
            jax.ShapeDtypeStruct((_NN, _D), f32))
  return pl.kernel(
      functools.partial(_layer_body, last),
      out_type=outs,
      mesh=_mesh,
      compiler_params=_cparams,
      scratch_types=[
          pltpu.VMEM((_NSUB, 128), i32),
          pltpu.VMEM((_MC,), i32),
          pltpu.VMEM((_NSUB, 128), i32),
          pltpu.VMEM((_L,), i32),
          pltpu.VMEM((_MC, _D), f32),
          pltpu.VMEM((_RQ, _D), f32),
          pltpu.VMEM((_RQ, _L), f32),
          pltpu.VMEM((_RQ, _D), f32),
          pltpu.VMEM((_RQ, _D), f32),
          pltpu.VMEM_SHARED((_ACC_ROWS, _D), f32),
          pltpu.SemaphoreType.DMA,
          pltpu.SemaphoreType.DMA,
      ],
  )


_part_kernel = _make_part()
_deg_kernel = _make_deg()
_layer_kernel = _make_layer(False)
_layer_kernel_last = _make_layer(True)


def kernel(user_emb, item_emb, edge_index):
  x0 = jnp.concatenate([user_emb, item_emb], axis=0)
  rows = edge_index[0]
  cols = edge_index[1]
  colp, dstp, cnt = _part_kernel(rows, cols)
  dinv, g0 = _deg_kernel(dstp, cnt, x0)
  s1, g1 = _layer_kernel(colp, dstp, cnt, dinv, g0, x0)
  s2, g2 = _layer_kernel(colp, dstp, cnt, dinv, g1, s1)
  out = _layer_kernel_last(colp, dstp, cnt, dinv, g2, s2)
  return (out[:_NU], out[_NU:])


# speedup vs baseline: 5.2921x; 5.2493x over previous
import functools

import jax
import jax.numpy as jnp
from jax import lax
from jax.experimental import pallas as pl
from jax.experimental.pallas import tpu as pltpu
from jax.experimental.pallas import tpu_sc as plsc

f32 = jnp.float32
i32 = jnp.int32

_NU = 25000
_NN = 50000
_D = 64
_E = 800000
_NC = 2
_NS = 16
_L = 16
_NQ = 4
_QSZ = _NN // _NQ
_TRASH = _QSZ
_ACC_ROWS = _QSZ + 8
_EPP = _E // (_NC * _NS)
_CAP = 26624
_PC = 256
_NPC = _EPP // _PC
_PTAIL = _EPP - _NPC * _PC
_MC = 1024
_NSUB = _MC // 128
_TQ = 832
_RQ = 32

_mesh = plsc.VectorSubcoreMesh(core_axis_name="c", subcore_axis_name="s")
_cparams = pltpu.CompilerParams(needs_layout_passes=False,
                                use_tc_tiling_on_sc=False)
_iota16 = lambda: lax.broadcasted_iota(i32, (_L,), 0)


def _rsqrt16(dv):
  ii = plsc.bitcast(dv, i32)
  ii = jnp.full((_L,), 0x5F3759DF, i32) - lax.shift_right_arithmetic(ii, 1)
  y = plsc.bitcast(ii, f32)
  for _ in range(3):
    y = y * (1.5 - 0.5 * dv * y * y)
  return jnp.where(dv > 0.0, y, 0.0)


def _zero_rows(buf, n):
  w = buf.shape[1]

  def body(i, _):
    for q in range(w // _L):
      buf[i, pl.ds(q * _L, _L)] = jnp.zeros((_L,), f32)
    return 0

  lax.fori_loop(0, n, body, 0)


def _count_for(cntbuf, q):
  v = cntbuf[pl.ds(0, _L)]
  return jnp.sum(jnp.where(_iota16() == q, v, 0))


def _part_body(rows_h, cols_h, colp_h, dstp_h, cnt_h,
               rbuf, cbuf, pend_c, pend_d, cntbuf):
  c = lax.axis_index("c")
  s = lax.axis_index("s")
  e0 = (s * _NC + c) * _EPP

  def do_group(offs, g, valid_mask):
    rv = rbuf[pl.ds(g * _L, _L)]
    cv = cbuf[pl.ds(g * _L, _L)]
    offs = list(offs)
    for q in range(_NQ):
      m = (rv >= q * _QSZ) & (rv < (q + 1) * _QSZ)
      if valid_mask is not None:
        m = m & valid_mask
      loc = rv - q * _QSZ
      plsc.store_compressed(pend_d.at[q, pl.ds(offs[q], _L)], loc, mask=m)
      plsc.store_compressed(pend_c.at[q, pl.ds(offs[q], _L)], cv, mask=m)
      offs[q] = offs[q] + plsc.all_reduce_population_count(m)[0]
    return tuple(offs)

  def flush_q(q, off, cnt):
    def yes(o, n):
      na = pl.multiple_of(n, 1024)
      pltpu.sync_copy(pend_c.at[q, pl.ds(0, _MC)],
                      colp_h.at[c, s, q, pl.ds(na, _MC)])
      pltpu.sync_copy(pend_d.at[q, pl.ds(0, _MC)],
                      dstp_h.at[c, s, q, pl.ds(na, _MC)])
      for g in range(_PC // _L):
        pend_c[q, pl.ds(g * _L, _L)] = pend_c[q, pl.ds(_MC + g * _L, _L)]
        pend_d[q, pl.ds(g * _L, _L)] = pend_d[q, pl.ds(_MC + g * _L, _L)]
      return o - _MC, n + _MC

    return lax.cond(off >= _MC, yes, lambda o, n: (o, n), off, cnt)

  def chunk(j, carry):
    pltpu.sync_copy(rows_h.at[pl.ds(e0 + j * _PC, _PC)], rbuf)
    pltpu.sync_copy(cols_h.at[pl.ds(e0 + j * _PC, _PC)], cbuf)
    offs = carry[:_NQ]
    cnts = carry[_NQ:]
    for g in range(_PC // _L):
      offs = do_group(offs, g, None)
    offs = list(offs)
    cnts = list(cnts)
    for q in range(_NQ):
      offs[q], cnts[q] = flush_q(q, offs[q], cnts[q])
    return tuple(offs) + tuple(cnts)

  zero = jnp.asarray(0, i32)
  carry = lax.fori_loop(0, _NPC, chunk, (zero,) * (2 * _NQ))
  pltpu.sync_copy(rows_h.at[pl.ds(e0 + _NPC * _PC, 176)],
                  rbuf.at[pl.ds(0, 176)])
  pltpu.sync_copy(cols_h.at[pl.ds(e0 + _NPC * _PC, 176)],
                  cbuf.at[pl.ds(0, 176)])
  offs = carry[:_NQ]
  cnts = list(carry[_NQ:])
  for g in range(_PTAIL // _L):
    offs = do_group(offs, g, None)
  tail_mask = _iota16() < (_PTAIL - (_PTAIL // _L) * _L)
  offs = do_group(offs, _PTAIL // _L, tail_mask)
  offs = list(offs)
  for q in range(_NQ):
    offs[q], cnts[q] = flush_q(q, offs[q], cnts[q])
  nv = jnp.zeros((_L,), i32)
  for q in range(_NQ):
    na = pl.multiple_of(cnts[q], 1024)
    pltpu.sync_copy(pend_c.at[q, pl.ds(0, _MC)],
                    colp_h.at[c, s, q, pl.ds(na, _MC)])
    pltpu.sync_copy(pend_d.at[q, pl.ds(0, _MC)],
                    dstp_h.at[c, s, q, pl.ds(na, _MC)])
    nv = jnp.where(_iota16() == q, jnp.full((_L,), cnts[q] + offs[q], i32), nv)
  cntbuf[pl.ds(0, _L)] = nv
  pltpu.sync_copy(cntbuf, cnt_h.at[c, s])


def _make_part():
  return pl.kernel(
      _part_body,
      out_type=(jax.ShapeDtypeStruct((_NC, _NS, _NQ, _CAP), i32),
                jax.ShapeDtypeStruct((_NC, _NS, _NQ, _CAP), i32),
                jax.ShapeDtypeStruct((_NC, _NS, _L), i32)),
      mesh=_mesh,
      compiler_params=_cparams,
      scratch_types=[
          pltpu.VMEM((_PC,), i32),
          pltpu.VMEM((_PC,), i32),
          pltpu.VMEM((_NQ, _MC + 2 * _PC), i32),
          pltpu.VMEM((_NQ, _MC + 2 * _PC), i32),
          pltpu.VMEM((_L,), i32),
      ],
  )


def _load_ibuf(dstp_h, c2, s2, q, off, dbuf, ibuf, sanitize=None, pad=_TRASH):
  pltpu.sync_copy(dstp_h.at[c2, s2, q, pl.ds(off, _MC)], dbuf)
  if sanitize is None:
    for g in range(_MC // _L):
      ibuf[g // 8, pl.ds((g % 8) * _L, _L)] = dbuf[pl.ds(g * _L, _L)]
  else:
    n, = sanitize
    for g in range(_MC // _L):
      eid = off + g * _L + _iota16()
      iv = dbuf[pl.ds(g * _L, _L)]
      pv = (g * _L + _iota16()) if pad is None else jnp.full((_L,), pad, i32)
      ibuf[g // 8, pl.ds((g % 8) * _L, _L)] = jnp.where(eid < n, iv, pv)


def _drain_cover(s):
  return jnp.minimum(s * _TQ, _QSZ - _TQ)


def _deg_body(dstp_h, cnt_h, x0_h, dinv_h, g0_h,
              ones_v, dibuf, ibuf, cntbuf, dbuf, dvbuf, xbuf, zb16, deg_sp,
              sems):
  c = lax.axis_index("c")
  s = lax.axis_index("s")

  def ones_fill(i, _):
    ones_v[i, :] = jnp.full((_L,), 1.0, f32)
    return 0

  lax.fori_loop(0, 128, ones_fill, 0)
  _zero_rows(zb16, _RQ)
  z0 = _drain_cover(s)

  def zloop(k, _):
    pltpu.sync_copy(zb16, deg_sp.at[pl.ds(z0 + k * _RQ, _RQ), :])
    return 0

  lax.fori_loop(0, _TQ // _RQ, zloop, 0)

  @pl.when(s == 0)
  def _():
    pltpu.sync_copy(zb16.at[pl.ds(0, 8), :], deg_sp.at[pl.ds(_QSZ, 8), :])

  plsc.subcore_barrier()

  def scatter8():
    descs = [
        pltpu.async_copy(ones_v, deg_sp.at[ibuf.at[k]], sems, add=True)
        for k in range(_NSUB)
    ]
    for d in descs:
      d.wait()

  for sweep in range(2):
    q = 2 * sweep + c
    for wpc in range(_NC):
      pltpu.sync_copy(cnt_h.at[wpc, s], cntbuf)
      n = _count_for(cntbuf, q)
      nmac = n // _MC
      t = n - nmac * _MC

      def mac(j, _):
        _load_ibuf(dstp_h, wpc, s, q, pl.multiple_of(j * _MC, 1024),
                   dibuf, ibuf)
        scatter8()
        return 0

      lax.fori_loop(0, nmac, mac, 0)

      @pl.when(t > 0)
      def _():
        _load_ibuf(dstp_h, wpc, s, q, pl.multiple_of(nmac * _MC, 1024),
                   dibuf, ibuf, sanitize=(n,))
        scatter8()

    plsc.subcore_barrier()
    r0 = _drain_cover(s)

    def dchunk(k, _):
      row0 = r0 + k * _RQ
      nbase = q * _QSZ + row0
      pltpu.sync_copy(deg_sp.at[pl.ds(row0, _RQ), :], dbuf)
      pltpu.sync_copy(x0_h.at[pl.ds(nbase, _RQ), :], xbuf)
      for i in range(_RQ):
        dvv = _rsqrt16(dbuf[i, :])
        dvbuf[i, :] = dvv
        for w in range(_D // _L):
          xbuf[i, pl.ds(w * _L, _L)] = xbuf[i, pl.ds(w * _L, _L)] * dvv
      pltpu.sync_copy(dvbuf, dinv_h.at[pl.ds(nbase, _RQ), :])
      pltpu.sync_copy(xbuf, g0_h.at[pl.ds(nbase, _RQ), :])
      return 0

    lax.fori_loop(0, _TQ // _RQ, dchunk, 0)
    if sweep == 0:
      plsc.subcore_barrier()
      lax.fori_loop(0, _TQ // _RQ, zloop, 0)
    plsc.subcore_barrier()


def _make_deg():
  return pl.kernel(
      _deg_body,
      out_type=(jax.ShapeDtypeStruct((_NN, _L), f32),
                jax.ShapeDtypeStruct((_NN, _D), f32)),
      mesh=_mesh,
      compiler_params=_cparams,
      scratch_types=[
          pltpu.VMEM((128, _L), f32),
          pltpu.VMEM((_MC,), i32),
          pltpu.VMEM((_NSUB, 128), i32),
          pltpu.VMEM((_L,), i32),
          pltpu.VMEM((_RQ, _L), f32),
          pltpu.VMEM((_RQ, _L), f32),
          pltpu.VMEM((_RQ, _D), f32),
          pltpu.VMEM((_RQ, _L), f32),
          pltpu.VMEM_SHARED((_ACC_ROWS, _L), f32),
          pltpu.SemaphoreType.DMA,
      ],
  )


def _layer_body(last, colp_h, dstp_h, cnt_h, dinv_h, g_h, sum_h, *refs):
  if last:
    (out_h, cibuf, dibuf, ibuf, cntbuf, gbuf, abuf, dvbuf, sbuf, zbuf, acc_sp,
     semg, sems) = refs
    go_h = None
  else:
    (out_h, go_h, cibuf, dibuf, ibuf, cntbuf, gbuf, abuf, dvbuf, sbuf, zbuf,
     acc_sp, semg, sems) = refs

  c = lax.axis_index("c")
  s = lax.axis_index("s")

  _zero_rows(zbuf, _RQ)
  z0 = _drain_cover(s)

  def zloop(k, _):
    pltpu.sync_copy(zbuf, acc_sp.at[pl.ds(z0 + k * _RQ, _RQ), :])
    return 0

  lax.fori_loop(0, _TQ // _RQ, zloop, 0)

  @pl.when(s == 0)
  def _():
    pltpu.sync_copy(zbuf.at[pl.ds(0, 8), :], acc_sp.at[pl.ds(_QSZ, 8), :])

  plsc.subcore_barrier()

  def run_macro():
    gd = [
        pltpu.async_copy(g_h.at[cibuf.at[k]],
                         gbuf.at[pl.ds(k * 128, 128), :], semg)
        for k in range(_NSUB)
    ]
    sd = []
    for k in range(_NSUB):
      gd[k].wait()
      sd.append(pltpu.async_copy(gbuf.at[pl.ds(k * 128, 128), :],
                                 acc_sp.at[ibuf.at[k]], sems, add=True))
    for d in sd:
      d.wait()

  for sweep in range(2):
    q = 2 * sweep + c
    for wpc in range(_NC):
      pltpu.sync_copy(cnt_h.at[wpc, s], cntbuf)
      n = _count_for(cntbuf, q)
      nmac = n // _MC
      t = n - nmac * _MC

      def mac(j, _):
        ja = pl.multiple_of(j * _MC, 1024)
        _load_ibuf(colp_h, wpc, s, q, ja, dibuf, cibuf)
        _load_ibuf(dstp_h, wpc, s, q, ja, dibuf, ibuf)
        run_macro()
        return 0

      lax.fori_loop(0, nmac, mac, 0)

      @pl.when(t > 0)
      def _():
        off = pl.multiple_of(nmac * _MC, 1024)
        _load_ibuf(colp_h, wpc, s, q, off, dibuf, cibuf, sanitize=(n,),
                   pad=None)
        _load_ibuf(dstp_h, wpc, s, q, off, dibuf, ibuf, sanitize=(n,))
        run_macro()

    plsc.subcore_barrier()
    r0 = _drain_cover(s)

    def dchunk(k, _):
      row0 = r0 + k * _RQ
      nbase = q * _QSZ + row0
      pltpu.sync_copy(acc_sp.at[pl.ds(row0, _RQ), :], abuf)
      pltpu.sync_copy(sum_h.at[pl.ds(nbase, _RQ), :], sbuf)
      pltpu.sync_copy(dinv_h.at[pl.ds(nbase, _RQ), :], dvbuf)

      def row(i, _):
        dvv = dvbuf[i, :]
        for w in range(_D // _L):
          xv = abuf[i, pl.ds(w * _L, _L)] * dvv
          sv = sbuf[i, pl.ds(w * _L, _L)] + xv
          if last:
            sbuf[i, pl.ds(w * _L, _L)] = sv * 0.25
          else:
            sbuf[i, pl.ds(w * _L, _L)] = sv
            abuf[i, pl.ds(w * _L, _L)] = xv * dvv
        return 0

      lax.fori_loop(0, _RQ, row, 0)
      pltpu.sync_copy(sbuf, out_h.at[pl.ds(nbase, _RQ), :])
      if not last:
        pltpu.sync_copy(abuf, go_h.at[pl.ds(nbase, _RQ), :])
      return 0

    lax.fori_loop(0, _TQ // _RQ, dchunk, 0)
    if sweep == 0:
      plsc.subcore_barrier()
      lax.fori_loop(0, _TQ // _RQ, zloop, 0)
      plsc.subcore_barrier()


def _make_layer(last):
  if last:
    outs = jax.ShapeDtypeStruct((_NN, _D), f32)
  else:
    outs = (jax.ShapeDtypeStruct((_NN, _D), f32),
            jax.ShapeDtypeStruct((_NN, _D), f32))
  return pl.kernel(
      functools.partial(_layer_body, last),
      out_type=outs,
      mesh=_mesh,
      compiler_params=_cparams,
      scratch_types=[
          pltpu.VMEM((_NSUB, 128), i32),
          pltpu.VMEM((_MC,), i32),
          pltpu.VMEM((_NSUB, 128), i32),
          pltpu.VMEM((_L,), i32),
          pltpu.VMEM((_MC, _D), f32),
          pltpu.VMEM((_RQ, _D), f32),
          pltpu.VMEM((_RQ, _L), f32),
          pltpu.VMEM((_RQ, _D), f32),
          pltpu.VMEM((_RQ, _D), f32),
          pltpu.VMEM_SHARED((_ACC_ROWS, _D), f32),
          pltpu.SemaphoreType.DMA,
          pltpu.SemaphoreType.DMA,
      ],
  )


_part_kernel = _make_part()
_deg_kernel = _make_deg()
_layer_kernel = _make_layer(False)
_layer_kernel_last = _make_layer(True)


def kernel(user_emb, item_emb, edge_index):
  x0 = jnp.concatenate([user_emb, item_emb], axis=0)
  rows = edge_index[0]
  cols = edge_index[1]
  colp, dstp, cnt = _part_kernel(rows, cols)
  dinv, g0 = _deg_kernel(dstp, cnt, x0)
  s1, g1 = _layer_kernel(colp, dstp, cnt, dinv, g0, x0)
  s2, g2 = _layer_kernel(colp, dstp, cnt, dinv, g1, s1)
  out = _layer_kernel_last(colp, dstp, cnt, dinv, g2, s2)
  return (out[:_NU], out[_NU:])
